# baseline (speedup 1.0000x reference)
import functools

import jax
import jax.numpy as jnp
from jax import lax
from jax.experimental import pallas as pl
from jax.experimental.pallas import tpu as pltpu
from jax.experimental.pallas import tpu_sc as plsc

VOCAB = 1000
D_MODEL = 64
BATCH = 1024
SEQ = 50
N_ROWS = BATCH * SEQ

_info = plsc.get_sparse_core_info()
NC, NS = _info.num_cores, _info.num_subcores
NW = NC * NS
ROWS_PER_W = N_ROWS // NW
CHUNK = 64
N_CHUNKS = ROWS_PER_W // CHUNK


def _mm_body(a_ref, b_ref, o_ref):
    o_ref[...] = lax.dot_general(
        a_ref[...], b_ref[...],
        (((1,), (1,)), ((), ())),
        preferred_element_type=jnp.float32,
    )


def _token_logit_table(embed_table, head_w):
    return pl.pallas_call(
        _mm_body,
        out_shape=jax.ShapeDtypeStruct((VOCAB, VOCAB), jnp.float32),
    )(embed_table, head_w)


_mesh = plsc.VectorSubcoreMesh(core_axis_name="c", subcore_axis_name="s")


@functools.partial(
    pl.kernel,
    mesh=_mesh,
    compiler_params=pltpu.CompilerParams(use_tc_tiling_on_sc=False),
    out_type=jax.ShapeDtypeStruct((N_ROWS, VOCAB), jnp.float32),
    scratch_types=[
        pltpu.VMEM((ROWS_PER_W,), jnp.int32),
        pltpu.VMEM((CHUNK, VOCAB), jnp.float32),
        pltpu.SemaphoreType.DMA,
    ],
)
def _gather_rows(m_hbm, idx_hbm, out_hbm, idx_v, buf, sem):
    wid = lax.axis_index("s") * NC + lax.axis_index("c")
    base = wid * ROWS_PER_W
    pltpu.sync_copy(idx_hbm.at[pl.ds(base, ROWS_PER_W)], idx_v)

    def body(c, carry):
        start = c * CHUNK
        pltpu.async_copy(
            m_hbm.at[idx_v.at[pl.ds(start, CHUNK)]], buf, sem
        ).wait()
        pltpu.sync_copy(buf, out_hbm.at[pl.ds(base + start, CHUNK)])
        return carry

    lax.fori_loop(0, N_CHUNKS, body, 0)


def kernel(input_ids, embed_table, head_w):
    m = _token_logit_table(embed_table, head_w)
    ids = input_ids.reshape(-1).astype(jnp.int32)
    out = _gather_rows(m, ids)
    return out.reshape(BATCH, SEQ, VOCAB)


# speedup vs baseline: 1.0137x; 1.0137x over previous
import functools

import jax
import jax.numpy as jnp
from jax import lax
from jax.experimental import pallas as pl
from jax.experimental.pallas import tpu as pltpu
from jax.experimental.pallas import tpu_sc as plsc

VOCAB = 1000
D_MODEL = 64
BATCH = 1024
SEQ = 50
N_ROWS = BATCH * SEQ

_info = plsc.get_sparse_core_info()
NC, NS = _info.num_cores, _info.num_subcores
NW = NC * NS
ROWS_PER_W = N_ROWS // NW
CHUNK = 40
N_CHUNKS = ROWS_PER_W // CHUNK


def _mm_body(a_ref, b_ref, o_ref):
    o_ref[...] = lax.dot_general(
        a_ref[...], b_ref[...],
        (((1,), (1,)), ((), ())),
        preferred_element_type=jnp.float32,
    )


def _token_logit_table(embed_table, head_w):
    return pl.pallas_call(
        _mm_body,
        out_shape=jax.ShapeDtypeStruct((VOCAB, VOCAB), jnp.float32),
    )(embed_table, head_w)


_mesh = plsc.VectorSubcoreMesh(core_axis_name="c", subcore_axis_name="s")


@functools.partial(
    pl.kernel,
    mesh=_mesh,
    compiler_params=pltpu.CompilerParams(use_tc_tiling_on_sc=False),
    out_type=jax.ShapeDtypeStruct((N_ROWS, VOCAB), jnp.float32),
    scratch_types=[
        pltpu.VMEM((ROWS_PER_W,), jnp.int32),
        pltpu.VMEM((CHUNK, VOCAB), jnp.float32),
        pltpu.VMEM((CHUNK, VOCAB), jnp.float32),
        pltpu.SemaphoreType.DMA,
        pltpu.SemaphoreType.DMA,
    ],
)
def _gather_rows(m_hbm, idx_hbm, out_hbm, idx_v, buf0, buf1, sem0, sem1):
    wid = lax.axis_index("s") * NC + lax.axis_index("c")
    base = wid * ROWS_PER_W
    pltpu.sync_copy(idx_hbm.at[pl.ds(base, ROWS_PER_W)], idx_v)

    def gather(c, buf, sem):
        return pltpu.make_async_copy(
            m_hbm.at[idx_v.at[pl.ds(c * CHUNK, CHUNK)]], buf, sem
        )

    def finish(c, buf, sem):
        gather(c, buf, sem).wait()
        pltpu.sync_copy(buf, out_hbm.at[pl.ds(base + c * CHUNK, CHUNK)])

    gather(0, buf0, sem0).start()

    def body(g, carry):
        c0 = 2 * g
        gather(c0 + 1, buf1, sem1).start()
        finish(c0, buf0, sem0)

        @pl.when(c0 + 2 < N_CHUNKS)
        def _():
            gather(c0 + 2, buf0, sem0).start()

        finish(c0 + 1, buf1, sem1)
        return carry

    lax.fori_loop(0, N_CHUNKS // 2, body, 0)


def kernel(input_ids, embed_table, head_w):
    m = _token_logit_table(embed_table, head_w)
    ids = input_ids.reshape(-1).astype(jnp.int32)
    out = _gather_rows(m, ids)
    return out.reshape(BATCH, SEQ, VOCAB)


# speedup vs baseline: 1.9595x; 1.9329x over previous
import functools

import jax
import jax.numpy as jnp
from jax import lax
from jax.experimental import pallas as pl
from jax.experimental.pallas import tpu as pltpu
from jax.experimental.pallas import tpu_sc as plsc

VOCAB = 1000
VPAD = 1024
D_MODEL = 64
BATCH = 1024
SEQ = 50
SEQ_PAD = 56

_info = plsc.get_sparse_core_info()
NC, NS = _info.num_cores, _info.num_subcores
NW = NC * NS
B_PER_W = BATCH // NW


def _mm_body(a_ref, b_ref, o_ref):
    o_ref[...] = lax.dot_general(
        a_ref[...], b_ref[...],
        (((1,), (1,)), ((), ())),
        preferred_element_type=jnp.float32,
    )


def _token_logit_table(embed_table, head_w_pad):
    return pl.pallas_call(
        _mm_body,
        out_shape=jax.ShapeDtypeStruct((VOCAB, VPAD), jnp.float32),
    )(embed_table, head_w_pad)


_mesh = plsc.VectorSubcoreMesh(core_axis_name="c", subcore_axis_name="s")


@functools.partial(
    pl.kernel,
    mesh=_mesh,
    out_type=jax.ShapeDtypeStruct((BATCH, SEQ, VPAD), jnp.float32),
    scratch_types=[
        pltpu.VMEM((B_PER_W * SEQ_PAD,), jnp.int32),
        pltpu.VMEM((SEQ, VPAD), jnp.float32),
        pltpu.SemaphoreType.DMA,
    ],
)
def _gather_rows(m_hbm, idx_hbm, out_hbm, idx_v, buf, sem):
    wid = lax.axis_index("s") * NC + lax.axis_index("c")
    pltpu.sync_copy(idx_hbm.at[pl.ds(wid * B_PER_W * SEQ_PAD, B_PER_W * SEQ_PAD)], idx_v)

    def body(j, carry):
        b = wid * B_PER_W + j
        pltpu.async_copy(
            m_hbm.at[idx_v.at[pl.ds(j * SEQ_PAD, SEQ)]], buf, sem
        ).wait()
        pltpu.sync_copy(buf, out_hbm.at[b])
        return carry

    lax.fori_loop(0, B_PER_W, body, 0)


def kernel(input_ids, embed_table, head_w):
    head_pad = jnp.pad(head_w, ((0, VPAD - VOCAB), (0, 0)))
    m = _token_logit_table(embed_table, head_pad)
    ids = jnp.pad(input_ids.astype(jnp.int32), ((0, 0), (0, SEQ_PAD - SEQ)))
    return _gather_rows(m, ids.reshape(-1))[:, :, :VOCAB]


# speedup vs baseline: 1.9948x; 1.0180x over previous
import functools

import jax
import jax.numpy as jnp
from jax import lax
from jax.experimental import pallas as pl
from jax.experimental.pallas import tpu as pltpu
from jax.experimental.pallas import tpu_sc as plsc

VOCAB = 1000
VPAD = 1024
NTC = VPAD // 128
D_MODEL = 64
BATCH = 1024
SEQ = 50
TPAD = 56

_info = plsc.get_sparse_core_info()
NC, NS = _info.num_cores, _info.num_subcores
NW = NC * NS
B_PER_W = BATCH // NW
IDX_PER_W = B_PER_W * NTC * TPAD


def _mm_body(a_ref, b_ref, o_ref):
    o_ref[...] = lax.dot_general(
        a_ref[...], b_ref[...],
        (((1,), (1,)), ((), ())),
        preferred_element_type=jnp.float32,
    )


def _token_logit_table(embed_table, head_w_pad):
    return pl.pallas_call(
        _mm_body,
        out_shape=jax.ShapeDtypeStruct((VOCAB, VPAD), jnp.float32),
    )(embed_table, head_w_pad)


_mesh = plsc.VectorSubcoreMesh(core_axis_name="c", subcore_axis_name="s")


@functools.partial(
    pl.kernel,
    mesh=_mesh,
    out_type=jax.ShapeDtypeStruct((BATCH, SEQ, VPAD), jnp.float32),
    scratch_types=[
        pltpu.VMEM((IDX_PER_W,), jnp.int32),
        pltpu.VMEM((SEQ, VPAD), jnp.float32),
        pltpu.VMEM((SEQ, VPAD), jnp.float32),
        pltpu.SemaphoreType.DMA,
        pltpu.SemaphoreType.DMA,
    ],
)
def _gather_rows(m8_hbm, idx_hbm, out_hbm, idx_v, buf0, buf1, sem0, sem1):
    wid = lax.axis_index("s") * NC + lax.axis_index("c")
    pltpu.sync_copy(idx_hbm.at[pl.ds(wid * IDX_PER_W, IDX_PER_W)], idx_v)

    def copies(j, buf, sem):
        return [
            pltpu.make_async_copy(
                m8_hbm.at[idx_v.at[pl.ds((j * NTC + tc) * TPAD, SEQ)]],
                buf.at[:, pl.ds(128 * tc, 128)],
                sem,
            )
            for tc in range(NTC)
        ]

    def start(j, buf, sem):
        for c in copies(j, buf, sem):
            c.start()

    def finish(j, buf, sem):
        for c in copies(j, buf, sem):
            c.wait()
        pltpu.sync_copy(buf, out_hbm.at[wid * B_PER_W + j])

    start(0, buf0, sem0)

    def body(g, carry):
        j0 = 2 * g
        start(j0 + 1, buf1, sem1)
        finish(j0, buf0, sem0)

        @pl.when(j0 + 2 < B_PER_W)
        def _():
            start(j0 + 2, buf0, sem0)

        finish(j0 + 1, buf1, sem1)
        return carry

    lax.fori_loop(0, B_PER_W // 2, body, 0)


def kernel(input_ids, embed_table, head_w):
    head_pad = jnp.pad(head_w, ((0, VPAD - VOCAB), (0, 0)))
    m = _token_logit_table(embed_table, head_pad)
    m8 = m.reshape(VOCAB * NTC, 128)
    ids = input_ids.astype(jnp.int32)
    idx_all = (NTC * ids)[:, None, :] + jnp.arange(NTC, dtype=jnp.int32)[None, :, None]
    idx_all = jnp.pad(idx_all, ((0, 0), (0, 0), (0, TPAD - SEQ)))
    return _gather_rows(m8, idx_all.reshape(-1))[:, :, :VOCAB]
